# baseline (speedup 1.0000x reference)
import functools

import jax
import jax.numpy as jnp
from jax import lax
from jax.experimental import pallas as pl
from jax.experimental.pallas import tpu as pltpu
from jax.experimental.pallas import tpu_sc as plsc

D = 1024
E = 8
H = 4096
N = 8192
TILE = 256
CAP = N * 2 + E * TILE
NTILES = CAP // TILE

NC = 2
NS = 16
NW = NC * NS
CHUNK = N // NW
SUB = 64
NSUB = CHUNK // SUB


def _topk_kernel(x_ref, wg_ref, bg_ref, wn_ref, bn_ref, eps_ref,
                 p1_ref, p2_ref, oh1_ref, oh2_ref):
    xt = x_ref[...]
    logits = jnp.dot(xt, wg_ref[...], preferred_element_type=jnp.float32) + bg_ref[...]
    noise = jax.nn.softplus(
        jnp.dot(xt, wn_ref[...], preferred_element_type=jnp.float32) + bn_ref[...])
    nl = logits + eps_ref[...] * noise
    lane = jax.lax.broadcasted_iota(jnp.int32, nl.shape, 1)
    v1 = jnp.max(nl, axis=-1, keepdims=True)
    i1 = jnp.min(jnp.where(nl == v1, lane, E), axis=-1, keepdims=True)
    nl2 = jnp.where(lane == i1, -jnp.inf, nl)
    v2 = jnp.max(nl2, axis=-1, keepdims=True)
    i2 = jnp.min(jnp.where(nl2 == v2, lane, E), axis=-1, keepdims=True)
    e2 = jnp.exp(v2 - v1)
    denom = 1.0 + e2
    p1_ref[...] = 1.0 / denom
    p2_ref[...] = e2 / denom
    oh1_ref[...] = (lane == i1).astype(jnp.float32)
    oh2_ref[...] = (lane == i2).astype(jnp.float32)


_G = 128
_NG = N // _G


def _dispatch_pos_kernel(oh1_ref, oh2_ref, pos1_ref, pos2_ref, texp_ref):
    counts1 = jnp.sum(oh1_ref[...], axis=0, keepdims=True)
    counts = counts1 + jnp.sum(oh2_ref[...], axis=0, keepdims=True)
    padded = jnp.ceil(counts * (1.0 / TILE)) * float(TILE)
    upper = (jax.lax.broadcasted_iota(jnp.int32, (E, E), 0)
             < jax.lax.broadcasted_iota(jnp.int32, (E, E), 1)).astype(jnp.float32)
    start = jnp.dot(padded, upper, preferred_element_type=jnp.float32)
    ltri = (jax.lax.broadcasted_iota(jnp.int32, (_G, _G), 0)
            >= jax.lax.broadcasted_iota(jnp.int32, (_G, _G), 1)).astype(jnp.float32)
    base1 = start
    base2 = start + counts1

    def body(g, run):
        run1, run2 = run
        sl = pl.ds(g * _G, _G)
        oh1 = oh1_ref[sl, :]
        oh2 = oh2_ref[sl, :]
        inc1 = jnp.dot(ltri, oh1, preferred_element_type=jnp.float32)
        inc2 = jnp.dot(ltri, oh2, preferred_element_type=jnp.float32)
        pos1_ref[sl, :] = jnp.sum(
            oh1 * (base1 + run1 + inc1 - oh1), axis=1,
            keepdims=True).astype(jnp.int32)
        pos2_ref[sl, :] = jnp.sum(
            oh2 * (base2 + run2 + inc2 - oh2), axis=1,
            keepdims=True).astype(jnp.int32)
        return (run1 + inc1[_G - 1:_G, :], run2 + inc2[_G - 1:_G, :])

    zero = jnp.zeros((1, E), jnp.float32)
    lax.fori_loop(0, _NG, body, (zero, zero))

    row = (jax.lax.broadcasted_iota(jnp.int32, (NTILES, E), 0)
           .astype(jnp.float32) * float(TILE))
    texp_ref[...] = (jnp.sum((row >= start).astype(jnp.int32), axis=1,
                             keepdims=True) - 1)


def _ffn_kernel(te_ref, xs_ref, w1_ref, b1_ref, w2_ref, b2_ref, ys_ref):
    del te_ref
    xb = xs_ref[...].astype(jnp.bfloat16)
    h = jnp.maximum(
        jnp.dot(xb, w1_ref[0], preferred_element_type=jnp.float32)
        + b1_ref[0], 0.0)
    ys_ref[...] = (
        jnp.dot(h.astype(jnp.bfloat16), w2_ref[0],
                preferred_element_type=jnp.float32)
        + b2_ref[0])


def _combine_kernel(p1_ref, p2_ref, y1_ref, y2_ref, out_ref):
    out_ref[...] = p1_ref[...] * y1_ref[...] + p2_ref[...] * y2_ref[...]


def _sc_mesh():
    return plsc.VectorSubcoreMesh(core_axis_name="c", subcore_axis_name="s")


def _dispatch_sc(xf, pos1, pos2):
    @functools.partial(
        pl.kernel,
        mesh=_sc_mesh(),
        out_type=jax.ShapeDtypeStruct((CAP, D), jnp.float32),
        scratch_types=[
            pltpu.VMEM((SUB, D), jnp.float32),
            pltpu.VMEM((SUB,), jnp.int32),
            pltpu.VMEM((SUB,), jnp.int32),
            pltpu.SemaphoreType.DMA,
        ],
    )
    def k(xf_hbm, pos1_hbm, pos2_hbm, xs_hbm, rows_v, idx1_v, idx2_v, sem):
        wid = lax.axis_index("s") * NC + lax.axis_index("c")
        base0 = wid * CHUNK

        def body(it, carry):
            base = base0 + it * SUB
            pltpu.sync_copy(xf_hbm.at[pl.ds(base, SUB)], rows_v)
            pltpu.sync_copy(pos1_hbm.at[pl.ds(base, SUB)], idx1_v)
            pltpu.sync_copy(pos2_hbm.at[pl.ds(base, SUB)], idx2_v)
            pltpu.async_copy(rows_v, xs_hbm.at[idx1_v], sem).wait()
            pltpu.async_copy(rows_v, xs_hbm.at[idx2_v], sem).wait()
            return carry

        lax.fori_loop(0, NSUB, body, 0)

    return k(xf, pos1, pos2)


def _collect_sc(ys, pos1, pos2):
    @functools.partial(
        pl.kernel,
        mesh=_sc_mesh(),
        out_type=(jax.ShapeDtypeStruct((N, D), jnp.float32),
                  jax.ShapeDtypeStruct((N, D), jnp.float32)),
        scratch_types=[
            pltpu.VMEM((SUB, D), jnp.float32),
            pltpu.VMEM((SUB,), jnp.int32),
            pltpu.SemaphoreType.DMA,
        ],
    )
    def k(ys_hbm, pos1_hbm, pos2_hbm, y1_hbm, y2_hbm, buf_v, idx_v, sem):
        wid = lax.axis_index("s") * NC + lax.axis_index("c")
        base0 = wid * CHUNK

        def body(it, carry):
            base = base0 + it * SUB
            pltpu.sync_copy(pos1_hbm.at[pl.ds(base, SUB)], idx_v)
            pltpu.async_copy(ys_hbm.at[idx_v], buf_v, sem).wait()
            pltpu.sync_copy(buf_v, y1_hbm.at[pl.ds(base, SUB)])
            pltpu.sync_copy(pos2_hbm.at[pl.ds(base, SUB)], idx_v)
            pltpu.async_copy(ys_hbm.at[idx_v], buf_v, sem).wait()
            pltpu.sync_copy(buf_v, y2_hbm.at[pl.ds(base, SUB)])
            return carry

        lax.fori_loop(0, NSUB, body, 0)

    return k(ys, pos1, pos2)


def kernel(x, Wg, bg, Wn, bn, W1, b1, W2, b2):
    B, S, _ = x.shape
    xf = x.reshape(N, D)
    eps = jax.random.normal(
        jax.random.key(42), (B, S, E), dtype=jnp.float32).reshape(N, E)

    RT = 1024
    p1, p2, oh1, oh2 = pl.pallas_call(
        _topk_kernel,
        grid=(N // RT,),
        in_specs=[
            pl.BlockSpec((RT, D), lambda i: (i, 0)),
            pl.BlockSpec((D, E), lambda i: (0, 0)),
            pl.BlockSpec((1, E), lambda i: (0, 0)),
            pl.BlockSpec((D, E), lambda i: (0, 0)),
            pl.BlockSpec((1, E), lambda i: (0, 0)),
            pl.BlockSpec((RT, E), lambda i: (i, 0)),
        ],
        out_specs=[
            pl.BlockSpec((RT, 1), lambda i: (i, 0)),
            pl.BlockSpec((RT, 1), lambda i: (i, 0)),
            pl.BlockSpec((RT, E), lambda i: (i, 0)),
            pl.BlockSpec((RT, E), lambda i: (i, 0)),
        ],
        out_shape=[
            jax.ShapeDtypeStruct((N, 1), jnp.float32),
            jax.ShapeDtypeStruct((N, 1), jnp.float32),
            jax.ShapeDtypeStruct((N, E), jnp.float32),
            jax.ShapeDtypeStruct((N, E), jnp.float32),
        ],
    )(xf, Wg, bg.reshape(1, E), Wn, bn.reshape(1, E), eps)

    pos1, pos2, texp = pl.pallas_call(
        _dispatch_pos_kernel,
        out_shape=[
            jax.ShapeDtypeStruct((N, 1), jnp.int32),
            jax.ShapeDtypeStruct((N, 1), jnp.int32),
            jax.ShapeDtypeStruct((NTILES, 1), jnp.int32),
        ],
    )(oh1, oh2)

    pos1f = pos1.reshape(N)
    pos2f = pos2.reshape(N)
    xs = _dispatch_sc(xf, pos1f, pos2f)

    texp_flat = texp.reshape(NTILES)
    ys = pl.pallas_call(
        _ffn_kernel,
        grid_spec=pltpu.PrefetchScalarGridSpec(
            num_scalar_prefetch=1,
            grid=(NTILES,),
            in_specs=[
                pl.BlockSpec((TILE, D), lambda i, te: (i, 0)),
                pl.BlockSpec((1, D, H), lambda i, te: (te[i], 0, 0)),
                pl.BlockSpec((1, 1, H), lambda i, te: (te[i], 0, 0)),
                pl.BlockSpec((1, H, D), lambda i, te: (te[i], 0, 0)),
                pl.BlockSpec((1, 1, D), lambda i, te: (te[i], 0, 0)),
            ],
            out_specs=pl.BlockSpec((TILE, D), lambda i, te: (i, 0)),
        ),
        out_shape=jax.ShapeDtypeStruct((CAP, D), jnp.float32),
    )(texp_flat, xs, W1.astype(jnp.bfloat16), b1.reshape(E, 1, H),
      W2.astype(jnp.bfloat16), b2.reshape(E, 1, D))

    y1, y2 = _collect_sc(ys, pos1f, pos2f)

    CT = 2048
    out = pl.pallas_call(
        _combine_kernel,
        grid=(N // CT,),
        in_specs=[
            pl.BlockSpec((CT, 1), lambda i: (i, 0)),
            pl.BlockSpec((CT, 1), lambda i: (i, 0)),
            pl.BlockSpec((CT, D), lambda i: (i, 0)),
            pl.BlockSpec((CT, D), lambda i: (i, 0)),
        ],
        out_specs=pl.BlockSpec((CT, D), lambda i: (i, 0)),
        out_shape=jax.ShapeDtypeStruct((N, D), jnp.float32),
    )(p1, p2, y1, y2)
    return out.reshape(B, S, D)


# speedup vs baseline: 1.0718x; 1.0718x over previous
import functools

import jax
import jax.numpy as jnp
from jax import lax
from jax.experimental import pallas as pl
from jax.experimental.pallas import tpu as pltpu
from jax.experimental.pallas import tpu_sc as plsc

D = 1024
E = 8
H = 4096
N = 8192
TILE = 256
CAP = N * 2 + E * TILE
NTILES = CAP // TILE

NC = 2
NS = 16
NW = NC * NS
CHUNK = N // NW
SUB = 64
NSUB = CHUNK // SUB


def _topk_kernel(x_ref, wg_ref, bg_ref, wn_ref, bn_ref, eps_ref,
                 p1_ref, p2_ref, oh1_ref, oh2_ref):
    xt = x_ref[...]
    logits = jnp.dot(xt, wg_ref[...], preferred_element_type=jnp.float32) + bg_ref[...]
    noise = jax.nn.softplus(
        jnp.dot(xt, wn_ref[...], preferred_element_type=jnp.float32) + bn_ref[...])
    nl = logits + eps_ref[...] * noise
    lane = jax.lax.broadcasted_iota(jnp.int32, nl.shape, 1)
    v1 = jnp.max(nl, axis=-1, keepdims=True)
    i1 = jnp.min(jnp.where(nl == v1, lane, E), axis=-1, keepdims=True)
    nl2 = jnp.where(lane == i1, -jnp.inf, nl)
    v2 = jnp.max(nl2, axis=-1, keepdims=True)
    i2 = jnp.min(jnp.where(nl2 == v2, lane, E), axis=-1, keepdims=True)
    e2 = jnp.exp(v2 - v1)
    denom = 1.0 + e2
    p1_ref[...] = 1.0 / denom
    p2_ref[...] = e2 / denom
    oh1_ref[...] = (lane == i1).astype(jnp.float32)
    oh2_ref[...] = (lane == i2).astype(jnp.float32)


_G = 1024
_NG = N // _G


def _dispatch_pos_kernel(oh1_ref, oh2_ref, pos1_ref, pos2_ref, texp_ref):
    counts1 = jnp.sum(oh1_ref[...], axis=0, keepdims=True)
    counts = counts1 + jnp.sum(oh2_ref[...], axis=0, keepdims=True)
    padded = jnp.ceil(counts * (1.0 / TILE)) * float(TILE)
    upper = (jax.lax.broadcasted_iota(jnp.int32, (E, E), 0)
             < jax.lax.broadcasted_iota(jnp.int32, (E, E), 1)).astype(jnp.float32)
    start = jnp.dot(padded, upper, preferred_element_type=jnp.float32)
    ltri = (jax.lax.broadcasted_iota(jnp.int32, (_G, _G), 0)
            >= jax.lax.broadcasted_iota(jnp.int32, (_G, _G), 1)).astype(jnp.float32)
    base1 = start
    base2 = start + counts1

    def body(g, run):
        run1, run2 = run
        sl = pl.ds(g * _G, _G)
        oh1 = oh1_ref[sl, :]
        oh2 = oh2_ref[sl, :]
        inc1 = jnp.dot(ltri, oh1, preferred_element_type=jnp.float32)
        inc2 = jnp.dot(ltri, oh2, preferred_element_type=jnp.float32)
        pos1_ref[sl, :] = jnp.sum(
            oh1 * (base1 + run1 + inc1 - oh1), axis=1,
            keepdims=True).astype(jnp.int32)
        pos2_ref[sl, :] = jnp.sum(
            oh2 * (base2 + run2 + inc2 - oh2), axis=1,
            keepdims=True).astype(jnp.int32)
        return (run1 + inc1[_G - 1:_G, :], run2 + inc2[_G - 1:_G, :])

    zero = jnp.zeros((1, E), jnp.float32)
    lax.fori_loop(0, _NG, body, (zero, zero))

    row = (jax.lax.broadcasted_iota(jnp.int32, (NTILES, E), 0)
           .astype(jnp.float32) * float(TILE))
    texp_ref[...] = (jnp.sum((row >= start).astype(jnp.int32), axis=1,
                             keepdims=True) - 1)


def _ffn_kernel(te_ref, xs_ref, ps_ref, w1_hbm, b1_ref, w2_hbm, b2_ref,
                ys_ref, w1_v, w2_v, sem1, sem2):
    i = pl.program_id(0)
    te_i = te_ref[i]
    changed = jnp.logical_or(
        i == 0, te_i != te_ref[jnp.maximum(i - 1, 0)])

    @pl.when(changed)
    def _():
        pltpu.make_async_copy(w1_hbm.at[te_i], w1_v, sem1).start()
        pltpu.make_async_copy(w2_hbm.at[te_i], w2_v, sem2).start()
        pltpu.make_async_copy(w1_hbm.at[te_i], w1_v, sem1).wait()

    h = jnp.maximum(
        jnp.dot(xs_ref[...], w1_v[...], preferred_element_type=jnp.float32)
        + b1_ref[0], 0.0)

    @pl.when(changed)
    def _():
        pltpu.make_async_copy(w2_hbm.at[te_i], w2_v, sem2).wait()

    ys_ref[...] = (
        jnp.dot(h, w2_v[...], preferred_element_type=jnp.float32)
        + b2_ref[0]) * ps_ref[...]


def _sc_mesh():
    return plsc.VectorSubcoreMesh(core_axis_name="c", subcore_axis_name="s")


def _dispatch_sc(xf, pos1, pos2, p1, p2):
    @functools.partial(
        pl.kernel,
        mesh=_sc_mesh(),
        out_type=(jax.ShapeDtypeStruct((CAP, D), jnp.float32),
                  jax.ShapeDtypeStruct((CAP,), jnp.float32)),
        scratch_types=[
            pltpu.VMEM((SUB, D), jnp.float32),
            pltpu.VMEM((SUB,), jnp.int32),
            pltpu.VMEM((SUB,), jnp.int32),
            pltpu.VMEM((SUB,), jnp.float32),
            pltpu.VMEM((SUB,), jnp.float32),
            pltpu.SemaphoreType.DMA,
        ],
    )
    def k(xf_hbm, pos1_hbm, pos2_hbm, p1_hbm, p2_hbm, xs_hbm, ps_hbm,
          rows_v, idx1_v, idx2_v, pv1_v, pv2_v, sem):
        wid = lax.axis_index("s") * NC + lax.axis_index("c")
        base0 = wid * CHUNK

        def body(it, carry):
            base = base0 + it * SUB
            sl = pl.ds(base, SUB)
            pltpu.sync_copy(xf_hbm.at[sl], rows_v)
            pltpu.sync_copy(pos1_hbm.at[sl], idx1_v)
            pltpu.sync_copy(pos2_hbm.at[sl], idx2_v)
            pltpu.sync_copy(p1_hbm.at[sl], pv1_v)
            pltpu.sync_copy(p2_hbm.at[sl], pv2_v)
            c1 = pltpu.async_copy(rows_v, xs_hbm.at[idx1_v], sem)
            c2 = pltpu.async_copy(rows_v, xs_hbm.at[idx2_v], sem)
            c3 = pltpu.async_copy(pv1_v, ps_hbm.at[idx1_v], sem)
            c4 = pltpu.async_copy(pv2_v, ps_hbm.at[idx2_v], sem)
            c1.wait()
            c2.wait()
            c3.wait()
            c4.wait()
            return carry

        lax.fori_loop(0, NSUB, body, 0)

    return k(xf, pos1, pos2, p1, p2)


def _collect_sc(ys, pos1, pos2):
    @functools.partial(
        pl.kernel,
        mesh=_sc_mesh(),
        out_type=jax.ShapeDtypeStruct((N, D), jnp.float32),
        scratch_types=[
            pltpu.VMEM((SUB, D), jnp.float32),
            pltpu.VMEM((SUB,), jnp.int32),
            pltpu.VMEM((SUB,), jnp.int32),
            pltpu.SemaphoreType.DMA,
        ],
    )
    def k(ys_hbm, pos1_hbm, pos2_hbm, out_hbm, buf_v, idx1_v, idx2_v, sem):
        wid = lax.axis_index("s") * NC + lax.axis_index("c")
        base0 = wid * CHUNK

        def body(it, carry):
            base = base0 + it * SUB
            sl = pl.ds(base, SUB)
            pltpu.sync_copy(pos1_hbm.at[sl], idx1_v)
            pltpu.sync_copy(pos2_hbm.at[sl], idx2_v)
            pltpu.async_copy(ys_hbm.at[idx1_v], buf_v, sem).wait()
            pltpu.async_copy(ys_hbm.at[idx2_v], buf_v, sem, add=True).wait()
            pltpu.sync_copy(buf_v, out_hbm.at[sl])
            return carry

        lax.fori_loop(0, NSUB, body, 0)

    return k(ys, pos1, pos2)


def kernel(x, Wg, bg, Wn, bn, W1, b1, W2, b2):
    B, S, _ = x.shape
    xf = x.reshape(N, D)
    eps = jax.random.normal(
        jax.random.key(42), (B, S, E), dtype=jnp.float32).reshape(N, E)

    RT = 1024
    p1, p2, oh1, oh2 = pl.pallas_call(
        _topk_kernel,
        grid=(N // RT,),
        in_specs=[
            pl.BlockSpec((RT, D), lambda i: (i, 0)),
            pl.BlockSpec((D, E), lambda i: (0, 0)),
            pl.BlockSpec((1, E), lambda i: (0, 0)),
            pl.BlockSpec((D, E), lambda i: (0, 0)),
            pl.BlockSpec((1, E), lambda i: (0, 0)),
            pl.BlockSpec((RT, E), lambda i: (i, 0)),
        ],
        out_specs=[
            pl.BlockSpec((RT, 1), lambda i: (i, 0)),
            pl.BlockSpec((RT, 1), lambda i: (i, 0)),
            pl.BlockSpec((RT, E), lambda i: (i, 0)),
            pl.BlockSpec((RT, E), lambda i: (i, 0)),
        ],
        out_shape=[
            jax.ShapeDtypeStruct((N, 1), jnp.float32),
            jax.ShapeDtypeStruct((N, 1), jnp.float32),
            jax.ShapeDtypeStruct((N, E), jnp.float32),
            jax.ShapeDtypeStruct((N, E), jnp.float32),
        ],
    )(xf, Wg, bg.reshape(1, E), Wn, bn.reshape(1, E), eps)

    pos1, pos2, texp = pl.pallas_call(
        _dispatch_pos_kernel,
        out_shape=[
            jax.ShapeDtypeStruct((N, 1), jnp.int32),
            jax.ShapeDtypeStruct((N, 1), jnp.int32),
            jax.ShapeDtypeStruct((NTILES, 1), jnp.int32),
        ],
    )(oh1, oh2)

    pos1f = pos1.reshape(N)
    pos2f = pos2.reshape(N)
    xs, ps = _dispatch_sc(xf, pos1f, pos2f, p1.reshape(N), p2.reshape(N))

    texp_flat = texp.reshape(NTILES)
    ys = pl.pallas_call(
        _ffn_kernel,
        grid_spec=pltpu.PrefetchScalarGridSpec(
            num_scalar_prefetch=1,
            grid=(NTILES,),
            in_specs=[
                pl.BlockSpec((TILE, D), lambda i, te: (i, 0)),
                pl.BlockSpec((TILE, 1), lambda i, te: (i, 0)),
                pl.BlockSpec(memory_space=pl.ANY),
                pl.BlockSpec((1, 1, H), lambda i, te: (te[i], 0, 0)),
                pl.BlockSpec(memory_space=pl.ANY),
                pl.BlockSpec((1, 1, D), lambda i, te: (te[i], 0, 0)),
            ],
            out_specs=pl.BlockSpec((TILE, D), lambda i, te: (i, 0)),
            scratch_shapes=[
                pltpu.VMEM((D, H), jnp.float32),
                pltpu.VMEM((H, D), jnp.float32),
                pltpu.SemaphoreType.DMA,
                pltpu.SemaphoreType.DMA,
            ],
        ),
        out_shape=jax.ShapeDtypeStruct((CAP, D), jnp.float32),
    )(texp_flat, xs, ps.reshape(CAP, 1), W1, b1.reshape(E, 1, H),
      W2, b2.reshape(E, 1, D))

    out = _collect_sc(ys, pos1f, pos2f)
    return out.reshape(B, S, D)
